# baseline (speedup 1.0000x reference)
import jax
import jax.numpy as jnp
from jax.experimental import pallas as pl
from jax.experimental.pallas import tpu as pltpu

_N_EMB = 1024
_DIM = 64
_R = 1024


def _vq_body(f_ref, embT_ref, embT2_ref, emb_ref, irow_ref,
             qT_ref, idx_ref, acc_ref, e2_ref):
    @pl.when(pl.program_id(0) == 0)
    def _init():
        emb = emb_ref[...]
        e2_ref[...] = jnp.sum(emb * emb, axis=1)[None, :]
        acc_ref[...] = jnp.zeros((1, 1), jnp.float32)

    f = f_ref[...]
    embT = embT_ref[...]

    f2 = jnp.sum(f * f, axis=1, keepdims=True)
    fe2 = jax.lax.dot_general(
        f, embT2_ref[...], (((1,), (0,)), ((), ())),
        preferred_element_type=jnp.float32)
    dist = (f2 + e2_ref[...]) - fe2

    dmin = jnp.min(dist, axis=1, keepdims=True)
    idxf = jnp.min(jnp.where(dist == dmin, irow_ref[...], jnp.float32(2.0 ** 30)),
                   axis=1)
    idx = idxf.astype(jnp.int32)
    idx_ref[0, 0, :] = idx

    ohT = (jax.lax.broadcasted_iota(jnp.int32, (_N_EMB, _R), 0)
           == idx[None, :]).astype(jnp.float32)
    qT = jax.lax.dot_general(
        embT, ohT, (((1,), (0,)), ((), ())),
        preferred_element_type=jnp.float32)
    qT_ref[0] = qT

    acc_ref[...] += jnp.sum(dmin)[None, None]


def kernel(z, embeddings):
    B, H, W, D = z.shape
    flat = z.reshape(-1, D)
    embT = embeddings.T
    embT2 = embT + embT
    iota_row = jax.lax.iota(jnp.float32, _N_EMB)[None, :]

    qT, idx, acc = pl.pallas_call(
        _vq_body,
        grid=(B,),
        in_specs=[
            pl.BlockSpec((_R, D), lambda b: (b, 0)),
            pl.BlockSpec((D, _N_EMB), lambda b: (0, 0)),
            pl.BlockSpec((D, _N_EMB), lambda b: (0, 0)),
            pl.BlockSpec((_N_EMB, D), lambda b: (0, 0)),
            pl.BlockSpec((1, _N_EMB), lambda b: (0, 0)),
        ],
        out_specs=[
            pl.BlockSpec((1, D, _R), lambda b: (b, 0, 0)),
            pl.BlockSpec((1, 1, _R), lambda b: (b, 0, 0)),
            pl.BlockSpec((1, 1), lambda b: (0, 0)),
        ],
        out_shape=[
            jax.ShapeDtypeStruct((B, D, _R), jnp.float32),
            jax.ShapeDtypeStruct((B, 1, _R), jnp.int32),
            jax.ShapeDtypeStruct((1, 1), jnp.float32),
        ],
        scratch_shapes=[pltpu.VMEM((1, _N_EMB), jnp.float32)],
    )(flat, embT, embT2, embeddings, iota_row)

    quantized_out = qT.reshape(B, D, H, W)
    indices = idx.reshape(B, 1, H, W)
    loss = acc[0, 0] * jnp.float32(0.25 / (B * H * W * D))
    return (quantized_out, indices, loss)


# speedup vs baseline: 1.1696x; 1.1696x over previous
import jax
import jax.numpy as jnp
from jax.experimental import pallas as pl
from jax.experimental.pallas import tpu as pltpu

_N_EMB = 1024
_DIM = 64
_R = 2048


def _vq_body(f_ref, embT_ref, emb_ref, irow_ref,
             qT_ref, idx_ref, acc_ref, e2_ref):
    @pl.when(pl.program_id(0) == 0)
    def _init():
        emb = emb_ref[...]
        e2_ref[...] = jnp.sum(emb * emb, axis=1)[None, :]
        acc_ref[...] = jnp.zeros((1, 1), jnp.float32)

    f = f_ref[...]
    embT = embT_ref[...]

    f2 = jnp.sum(f * f, axis=1, keepdims=True)
    fe = jax.lax.dot_general(
        f, embT, (((1,), (0,)), ((), ())),
        preferred_element_type=jnp.float32)
    dist = (f2 + e2_ref[...]) - 2.0 * fe

    dmin = jnp.min(dist, axis=1, keepdims=True)
    idxf = jnp.min(jnp.where(dist == dmin, irow_ref[...], jnp.float32(2.0 ** 30)),
                   axis=1)
    idx = idxf.astype(jnp.int32)
    idx_ref[0, 0, :] = idx[:1024]
    idx_ref[1, 0, :] = idx[1024:]

    ohT = (jax.lax.broadcasted_iota(jnp.int32, (_N_EMB, _R), 0)
           == idx[None, :]).astype(jnp.float32)
    qT = jax.lax.dot_general(
        embT, ohT, (((1,), (0,)), ((), ())),
        preferred_element_type=jnp.float32)
    qT_ref[0] = qT[:, :1024]
    qT_ref[1] = qT[:, 1024:]

    acc_ref[...] += jnp.sum(dmin)[None, None]


def kernel(z, embeddings):
    B, H, W, D = z.shape
    flat = z.reshape(-1, D)
    embT = embeddings.T
    iota_row = jax.lax.iota(jnp.float32, _N_EMB)[None, :]

    qT, idx, acc = pl.pallas_call(
        _vq_body,
        grid=(B // 2,),
        in_specs=[
            pl.BlockSpec((_R, D), lambda b: (b, 0)),
            pl.BlockSpec((D, _N_EMB), lambda b: (0, 0)),
            pl.BlockSpec((_N_EMB, D), lambda b: (0, 0)),
            pl.BlockSpec((1, _N_EMB), lambda b: (0, 0)),
        ],
        out_specs=[
            pl.BlockSpec((2, D, 1024), lambda b: (b, 0, 0)),
            pl.BlockSpec((2, 1, 1024), lambda b: (b, 0, 0)),
            pl.BlockSpec((1, 1), lambda b: (0, 0)),
        ],
        out_shape=[
            jax.ShapeDtypeStruct((B, D, 1024), jnp.float32),
            jax.ShapeDtypeStruct((B, 1, 1024), jnp.int32),
            jax.ShapeDtypeStruct((1, 1), jnp.float32),
        ],
        scratch_shapes=[pltpu.VMEM((1, _N_EMB), jnp.float32)],
    )(flat, embT, embeddings, iota_row)

    quantized_out = qT.reshape(B, D, H, W)
    indices = idx.reshape(B, 1, H, W)
    loss = acc[0, 0] * jnp.float32(0.25 / (B * H * W * D))
    return (quantized_out, indices, loss)
